# baseline (speedup 1.0000x reference)
import functools

import jax
import jax.numpy as jnp
from jax import lax
from jax.experimental import pallas as pl
from jax.experimental.pallas import tpu as pltpu
from jax.experimental.pallas import tpu_sc as plsc

_NBUF = 2
_G = 2


@functools.cache
def _build(n, s, V, D):
    info = plsc.get_sparse_core_info()
    nw = info.num_cores * info.num_subcores
    IC = n // 128
    DT = D // 8
    blocks = s * IC
    per_w = blocks // nw
    chunks = per_w // _G
    assert chunks % _NBUF == 0 and IC % _G == 0 and per_w % _G == 0
    CT = _G * 128
    SL = _G * 1024

    mesh = plsc.VectorSubcoreMesh(core_axis_name="c", subcore_axis_name="s")

    @functools.partial(
        pl.kernel,
        mesh=mesh,
        out_type=jax.ShapeDtypeStruct((n * s * D,), jnp.float32),
        compiler_params=pltpu.CompilerParams(
            use_tc_tiling_on_sc=False, needs_layout_passes=False
        ),
        scratch_types=[
            pltpu.VMEM((_NBUF, CT), jnp.int32),
            pltpu.VMEM((_NBUF, CT, 128), jnp.float32),
            pltpu.VMEM((_NBUF, DT, SL), jnp.float32),
            pltpu.SemaphoreType.DMA((_NBUF,)),
            pltpu.SemaphoreType.DMA((_NBUF,)),
        ],
    )
    def kern(tid_hbm, wp_hbm, out_hbm, idx_v, rows_v, slab_v, gsem, wsem):
        wid = lax.axis_index("s") * info.num_cores + lax.axis_index("c")
        beta0w = wid * per_w

        def load_idx_and_fire(c, b):
            beta0 = beta0w + _G * c
            j = beta0 // IC
            ic0 = beta0 % IC
            pltpu.sync_copy(
                tid_hbm.at[pl.ds(j * n + ic0 * 128, CT)], idx_v.at[b]
            )
            for k in range(_G):
                pltpu.async_copy(
                    wp_hbm.at[idx_v.at[b, pl.ds(k * 128, 128)]],
                    rows_v.at[b, pl.ds(k * 128, 128)],
                    gsem.at[b],
                )

        def drain_gather(b):
            pltpu.make_async_copy(
                wp_hbm.at[pl.ds(0, CT)], rows_v.at[b], gsem.at[b]
            ).wait()

        def wait_writes(b):
            for dt in range(DT):
                pltpu.make_async_copy(
                    slab_v.at[b, dt], out_hbm.at[pl.ds(0, SL)], wsem.at[b]
                ).wait()

        lanes = lax.iota(jnp.int32, 16)

        def transpose(b):
            @pl.loop(0, _G * DT)
            def _m(m):
                k = m // DT
                dt = m % DT
                for ds in range(8):
                    col = jnp.full((16,), 0, jnp.int32) + (dt * 8 + ds)
                    for grp in range(8):
                        row = lanes + (k * 128 + grp * 16)
                        v = plsc.load_gather(rows_v.at[b], [row, col])
                        slab_v[
                            b, dt, pl.ds(k * 1024 + ds * 128 + grp * 16, 16)
                        ] = v

        def fire_writes(c, b):
            beta0 = beta0w + _G * c
            j = beta0 // IC
            ic0 = beta0 % IC
            for dt in range(DT):
                base = ((j * DT + dt) * IC + ic0) * 1024
                pltpu.async_copy(
                    slab_v.at[b, dt], out_hbm.at[pl.ds(base, SL)], wsem.at[b]
                )

        for b in range(_NBUF):
            load_idx_and_fire(b, b)

        @pl.loop(0, chunks, step=_NBUF)
        def _chunks(c0):
            for b in range(_NBUF):
                c = c0 + b
                drain_gather(b)

                @pl.when(c >= _NBUF)
                def _():
                    wait_writes(b)

                transpose(b)
                fire_writes(c, b)

                @pl.when(c + _NBUF < chunks)
                def _():
                    load_idx_and_fire(c + _NBUF, b)

        for b in range(_NBUF):
            wait_writes(b)

    return kern


def kernel(token_ids, weight):
    n, s = token_ids.shape
    V, D = weight.shape
    sp = -s % 8
    tid1 = jnp.pad(token_ids.T, ((0, sp), (0, 0))).reshape((s + sp) * n)
    wp = jnp.pad(weight, ((0, 0), (0, 128 - D)))
    out1 = _build(n, s, V, D)(tid1, wp)
    out5 = out1.reshape(s, D // 8, n // 128, 8, 128)
    out5 = out5.transpose(0, 1, 3, 2, 4)
    return out5.reshape(s, D, n).transpose(2, 0, 1)
